# baseline (speedup 1.0000x reference)
import jax
import jax.numpy as jnp
from jax import lax
from jax.experimental import pallas as pl
from jax.experimental.pallas import tpu as pltpu
from jax.experimental.pallas import tpu_sc as plsc

_B, _S, _H = 4, 2048, 768
_EPS = 1e-5
_L = 16
_HV = _H // _L
_NC, _NS = 2, 16
_NW = _NC * _NS
_CH = _S // _NW
_TS = 8
_NSUB = _CH // _TS
_GR = _B * _TS
_HU = 12


def _allreduce_lanes(v):
    lanes = lax.iota(jnp.int32, _L)
    dnums = lax.GatherDimensionNumbers(
        offset_dims=(), collapsed_slice_dims=(0,), start_index_map=(0,))
    for shift in (8, 4, 2, 1):
        perm = lax.bitwise_xor(lanes, jnp.int32(shift))
        v = v + lax.gather(
            v, perm[:, None], dnums, slice_sizes=(1,),
            mode=lax.GatherScatterMode.PROMISE_IN_BOUNDS)
    return v


def _rsqrt(x):
    i = lax.bitcast_convert_type(x, jnp.int32)
    y = lax.bitcast_convert_type(
        jnp.int32(0x5F3759DF) - lax.shift_right_arithmetic(i, 1), jnp.float32)
    half_x = x * 0.5
    for _ in range(3):
        y = y * (1.5 - half_x * y * y)
    return y


def _compute_chunk(gbuf, wbuf, pbuf, type_v, gamma_v, beta_v):
    inv_h = jnp.float32(1.0 / _H)
    z = jnp.zeros((_L,), jnp.float32)

    def row_body(t, _):
        pb = t * _H

        @plsc.parallel_loop(0, _HV, unroll=_HU, carry=(z,) * (2 * _B))
        def accs(hv, carry):
            accs = list(carry)
            ho = hv * _L
            pv = pbuf[pl.ds(pb + ho, _L)] + type_v[pl.ds(ho, _L)]
            for b in range(_B):
                e = gbuf[b * _TS + t, pl.ds(ho, _L)] + pv
                gbuf[b * _TS + t, pl.ds(ho, _L)] = e
                accs[2 * b] = accs[2 * b] + e
                accs[2 * b + 1] = accs[2 * b + 1] + e * e
            return tuple(accs)

        means = []
        ys = []
        for b in range(_B):
            mean_v = _allreduce_lanes(accs[2 * b]) * inv_h
            var_v = _allreduce_lanes(accs[2 * b + 1]) * inv_h - mean_v * mean_v
            means.append(mean_v)
            ys.append(_rsqrt(var_v + _EPS))

        @plsc.parallel_loop(0, _HV, unroll=_HU)
        def _(hv):
            ho = hv * _L
            g = gamma_v[pl.ds(ho, _L)]
            bt = beta_v[pl.ds(ho, _L)]
            for b in range(_B):
                e = gbuf[b * _TS + t, pl.ds(ho, _L)]
                wbuf[b * _TS + t, pl.ds(ho, _L)] = \
                    (e - means[b]) * ys[b] * g + bt

        return 0

    lax.fori_loop(0, _TS, row_body, 0)


def _emb_kernel(ids_hbm, wemb_hbm, pos_hbm, type_hbm, gamma_hbm, beta_hbm,
                out_hbm,
                idx_v, g0, g1, w0, w1, p0, p1, type_v, gamma_v, beta_v,
                semg, semp, semw):
    wid = lax.axis_index("s") * _NC + lax.axis_index("c")
    s0 = wid * _CH

    gbufs, wbufs, pbufs = (g0, g1), (w0, w1), (p0, p1)

    pltpu.sync_copy(ids_hbm.at[pl.ds(wid * (_NSUB * _GR), _NSUB * _GR)], idx_v)
    pltpu.sync_copy(type_hbm, type_v)
    pltpu.sync_copy(gamma_hbm, gamma_v)
    pltpu.sync_copy(beta_hbm, beta_v)

    def fire_gather(sub, par):
        pltpu.async_copy(
            wemb_hbm.at[idx_v.at[pl.ds(sub * _GR, _GR)]], gbufs[par], semg)
        pltpu.async_copy(
            pos_hbm.at[pl.ds((s0 + sub * _TS) * _H, _TS * _H)],
            pbufs[par], semp)

    def wait_gather(par):
        pltpu.make_async_copy(
            wemb_hbm.at[idx_v.at[pl.ds(0, _GR)]], gbufs[par], semg).wait()
        pltpu.make_async_copy(
            pos_hbm.at[pl.ds(0, _TS * _H)], pbufs[par], semp).wait()

    def fire_wb(sub, par):
        for b in range(_B):
            pltpu.async_copy(
                gbufs[par].at[pl.ds(b * _TS, _TS)],
                out_hbm.at[pl.ds(b * _S + s0 + sub * _TS, _TS)], semw)

    def wait_wb(par):
        for b in range(_B):
            pltpu.make_async_copy(
                wbufs[par].at[pl.ds(b * _TS, _TS)],
                out_hbm.at[pl.ds(b * _S, _TS)], semw).wait()

    fire_gather(0, 0)
    wait_gather(0)

    def step(sub, par):
        _compute_chunk(gbufs[par], wbufs[par], pbufs[par],
                       type_v, gamma_v, beta_v)

    def loop_body(sub, _):
        step(sub, 0)
        step(sub + 1, 1)
        return 0

    lax.fori_loop(0, _NSUB // 2, lambda i, c: loop_body(i * 2, c), 0,
                  unroll=False)
    fire_wb(0, 0)
    fire_wb(0, 1)
    wait_wb(0)
    wait_wb(1)


def kernel(input_ids, word_emb, pos_emb, type_emb, gamma, beta):
    ids_re = (input_ids.reshape(_B, _NW, _NSUB, _TS)
              .transpose(1, 2, 0, 3).reshape(-1))
    pos_flat = pos_emb[:_S].reshape(_S * _H)
    type_row = type_emb.reshape(-1)[:_H]

    mesh = plsc.VectorSubcoreMesh(core_axis_name="c", subcore_axis_name="s")
    run = pl.kernel(
        _emb_kernel,
        out_type=jax.ShapeDtypeStruct((_B * _S, _H), jnp.float32),
        mesh=mesh,
        scratch_types=[
            pltpu.VMEM((_NSUB * _GR,), jnp.int32),
            pltpu.VMEM((_GR, _H), jnp.float32),
            pltpu.VMEM((_GR, _H), jnp.float32),
            pltpu.VMEM((_GR, _H), jnp.float32),
            pltpu.VMEM((_GR, _H), jnp.float32),
            pltpu.VMEM((_TS * _H,), jnp.float32),
            pltpu.VMEM((_TS * _H,), jnp.float32),
            pltpu.VMEM((_H,), jnp.float32),
            pltpu.VMEM((_H,), jnp.float32),
            pltpu.VMEM((_H,), jnp.float32),
            pltpu.SemaphoreType.DMA,
            pltpu.SemaphoreType.DMA,
            pltpu.SemaphoreType.DMA,
        ],
    )
    out = run(ids_re, word_emb, pos_flat, type_row, gamma, beta)
    return out.reshape(_B, _S, _H)


# speedup vs baseline: 1.1286x; 1.1286x over previous
import jax
import jax.numpy as jnp
from jax import lax
from jax.experimental import pallas as pl
from jax.experimental.pallas import tpu as pltpu
from jax.experimental.pallas import tpu_sc as plsc

_B, _S, _H = 4, 2048, 768
_EPS = 1e-5
_L = 16
_HV = _H // _L
_NC, _NS = 2, 16
_NW = _NC * _NS
_CH = _S // _NW
_TS = 8
_NSUB = _CH // _TS
_GR = _B * _TS
_HU = 48


def _allreduce_lanes(v):
    lanes = lax.iota(jnp.int32, _L)
    dnums = lax.GatherDimensionNumbers(
        offset_dims=(), collapsed_slice_dims=(0,), start_index_map=(0,))
    for shift in (8, 4, 2, 1):
        perm = lax.bitwise_xor(lanes, jnp.int32(shift))
        v = v + lax.gather(
            v, perm[:, None], dnums, slice_sizes=(1,),
            mode=lax.GatherScatterMode.PROMISE_IN_BOUNDS)
    return v


def _rsqrt(x):
    i = lax.bitcast_convert_type(x, jnp.int32)
    y = lax.bitcast_convert_type(
        jnp.int32(0x5F3759DF) - lax.shift_right_arithmetic(i, 1), jnp.float32)
    half_x = x * 0.5
    for _ in range(3):
        y = y * (1.5 - half_x * y * y)
    return y


def _compute_chunk(gbuf, wbuf, pbuf, type_v, gamma_v, beta_v):
    inv_h = jnp.float32(1.0 / _H)
    z = jnp.zeros((_L,), jnp.float32)

    def row_body(t, _):
        pb = t * _H

        @plsc.parallel_loop(0, _HV, unroll=_HU, carry=(z,) * (2 * _B))
        def accs(hv, carry):
            accs = list(carry)
            ho = hv * _L
            pv = pbuf[pl.ds(pb + ho, _L)] + type_v[pl.ds(ho, _L)]
            for b in range(_B):
                e = gbuf[b * _TS + t, pl.ds(ho, _L)] + pv
                gbuf[b * _TS + t, pl.ds(ho, _L)] = e
                accs[2 * b] = accs[2 * b] + e
                accs[2 * b + 1] = accs[2 * b + 1] + e * e
            return tuple(accs)

        means = []
        ys = []
        for b in range(_B):
            mean_v = _allreduce_lanes(accs[2 * b]) * inv_h
            var_v = _allreduce_lanes(accs[2 * b + 1]) * inv_h - mean_v * mean_v
            means.append(mean_v)
            ys.append(_rsqrt(var_v + _EPS))

        @plsc.parallel_loop(0, _HV, unroll=_HU)
        def _(hv):
            ho = hv * _L
            g = gamma_v[pl.ds(ho, _L)]
            bt = beta_v[pl.ds(ho, _L)]
            for b in range(_B):
                e = gbuf[b * _TS + t, pl.ds(ho, _L)]
                wbuf[b * _TS + t, pl.ds(ho, _L)] = \
                    (e - means[b]) * ys[b] * g + bt

        return 0

    lax.fori_loop(0, _TS, row_body, 0)


def _emb_kernel(ids_hbm, wemb_hbm, pos_hbm, type_hbm, gamma_hbm, beta_hbm,
                out_hbm,
                idx_v, g0, g1, w0, w1, p0, p1, type_v, gamma_v, beta_v,
                semg, semp, semw):
    wid = lax.axis_index("s") * _NC + lax.axis_index("c")
    s0 = wid * _CH

    gbufs, wbufs, pbufs = (g0, g1), (w0, w1), (p0, p1)

    pltpu.sync_copy(ids_hbm.at[pl.ds(wid * (_NSUB * _GR), _NSUB * _GR)], idx_v)
    pltpu.sync_copy(type_hbm, type_v)
    pltpu.sync_copy(gamma_hbm, gamma_v)
    pltpu.sync_copy(beta_hbm, beta_v)

    def fire_gather(sub, par):
        pltpu.async_copy(
            wemb_hbm.at[idx_v.at[pl.ds(sub * _GR, _GR)]], gbufs[par], semg)
        pltpu.async_copy(
            pos_hbm.at[pl.ds((s0 + sub * _TS) * _H, _TS * _H)],
            pbufs[par], semp)

    def wait_gather(par):
        pltpu.make_async_copy(
            wemb_hbm.at[idx_v.at[pl.ds(0, _GR)]], gbufs[par], semg).wait()
        pltpu.make_async_copy(
            pos_hbm.at[pl.ds(0, _TS * _H)], pbufs[par], semp).wait()

    def fire_wb(sub, par):
        for b in range(_B):
            pltpu.async_copy(
                wbufs[par].at[pl.ds(b * _TS, _TS)],
                out_hbm.at[pl.ds(b * _S + s0 + sub * _TS, _TS)], semw)

    def wait_wb(par):
        for b in range(_B):
            pltpu.make_async_copy(
                wbufs[par].at[pl.ds(b * _TS, _TS)],
                out_hbm.at[pl.ds(b * _S, _TS)], semw).wait()

    fire_gather(0, 0)

    def step(sub, par):
        @pl.when(sub < _NSUB - 1)
        def _():
            fire_gather(sub + 1, 1 - par)

        @pl.when(sub >= 2)
        def _():
            wait_wb(par)
        wait_gather(par)
        _compute_chunk(gbufs[par], wbufs[par], pbufs[par],
                       type_v, gamma_v, beta_v)
        fire_wb(sub, par)

    def loop_body(sub, _):
        step(sub, 0)
        step(sub + 1, 1)
        return 0

    lax.fori_loop(0, _NSUB // 2, lambda i, c: loop_body(i * 2, c), 0,
                  unroll=False)
    wait_wb(0)
    wait_wb(1)


def kernel(input_ids, word_emb, pos_emb, type_emb, gamma, beta):
    ids_re = (input_ids.reshape(_B, _NW, _NSUB, _TS)
              .transpose(1, 2, 0, 3).reshape(-1))
    pos_flat = pos_emb[:_S].reshape(_S * _H)
    type_row = type_emb.reshape(-1)[:_H]

    mesh = plsc.VectorSubcoreMesh(core_axis_name="c", subcore_axis_name="s")
    run = pl.kernel(
        _emb_kernel,
        out_type=jax.ShapeDtypeStruct((_B * _S, _H), jnp.float32),
        mesh=mesh,
        scratch_types=[
            pltpu.VMEM((_NSUB * _GR,), jnp.int32),
            pltpu.VMEM((_GR, _H), jnp.float32),
            pltpu.VMEM((_GR, _H), jnp.float32),
            pltpu.VMEM((_GR, _H), jnp.float32),
            pltpu.VMEM((_GR, _H), jnp.float32),
            pltpu.VMEM((_TS * _H,), jnp.float32),
            pltpu.VMEM((_TS * _H,), jnp.float32),
            pltpu.VMEM((_H,), jnp.float32),
            pltpu.VMEM((_H,), jnp.float32),
            pltpu.VMEM((_H,), jnp.float32),
            pltpu.SemaphoreType.DMA,
            pltpu.SemaphoreType.DMA,
            pltpu.SemaphoreType.DMA,
        ],
    )
    out = run(ids_re, word_emb, pos_flat, type_row, gamma, beta)
    return out.reshape(_B, _S, _H)


# speedup vs baseline: 1.5129x; 1.3405x over previous
import functools

import jax
import jax.numpy as jnp
from jax import lax
from jax.experimental import pallas as pl
from jax.experimental.pallas import tpu as pltpu
from jax.experimental.pallas import tpu_sc as plsc

_B, _S, _H = 4, 2048, 768
_EPS = 1e-5
_NC, _NS = 2, 16
_NW = _NC * _NS
_CH = _S // _NW
_RB = 256
_NRB = _S // _RB


def _gather_kernel(ids_hbm, wemb_hbm, out_hbm, idx_v, buf_v, sem):
    wid = lax.axis_index("s") * _NC + lax.axis_index("c")
    base = wid * _CH
    pltpu.sync_copy(ids_hbm.at[pl.ds(base, _CH)], idx_v)
    pltpu.async_copy(wemb_hbm.at[idx_v], buf_v, sem).wait()
    pltpu.sync_copy(buf_v, out_hbm.at[pl.ds(base, _CH)])


def _sc_gather(ids_b, word_emb):
    mesh = plsc.VectorSubcoreMesh(core_axis_name="c", subcore_axis_name="s")
    run = pl.kernel(
        _gather_kernel,
        out_type=jax.ShapeDtypeStruct((_S, _H), jnp.float32),
        mesh=mesh,
        scratch_types=[
            pltpu.VMEM((_CH,), jnp.int32),
            pltpu.VMEM((_CH, _H), jnp.float32),
            pltpu.SemaphoreType.DMA,
        ],
    )
    return run(ids_b, word_emb)


def _ln_body(buf_ref, g_ref, p_ref, t_ref, gamma_ref, beta_ref, o_ref):
    x = g_ref[...] + p_ref[...] + t_ref[0][None, :]
    mean = jnp.mean(x, axis=-1, keepdims=True)
    xc = x - mean
    var = jnp.mean(xc * xc, axis=-1, keepdims=True)
    o_ref[0] = xc * lax.rsqrt(var + _EPS) * gamma_ref[...] + beta_ref[...]


def _tc_ln(b, buf, g_b, pos, type_row, gamma, beta):
    grid = (_NRB,)
    in_specs = [
        pl.BlockSpec((1, 8, 128), lambda i: (0, 0, 0)),
        pl.BlockSpec((_RB, _H), lambda i: (i, 0)),
        pl.BlockSpec((_RB, _H), lambda i: (i, 0)),
        pl.BlockSpec((1, _H), lambda i: (0, 0)),
        pl.BlockSpec((_H,), lambda i: (0,)),
        pl.BlockSpec((_H,), lambda i: (0,)),
    ]
    out_spec = pl.BlockSpec((1, _RB, _H), lambda i, _b=b: (_b, i, 0))
    return pl.pallas_call(
        _ln_body,
        grid=grid,
        in_specs=in_specs,
        out_specs=out_spec,
        out_shape=jax.ShapeDtypeStruct((_B, _S, _H), jnp.float32),
        input_output_aliases={0: 0},
    )(buf, g_b, pos, type_row, gamma, beta)


def kernel(input_ids, word_emb, pos_emb, type_emb, gamma, beta):
    pos = pos_emb[:_S]
    type_row = type_emb.reshape(1, -1)[:, :_H]

    gathered = [_sc_gather(input_ids[b], word_emb) for b in range(_B)]

    buf = jnp.zeros((_B, _S, _H), jnp.float32)
    for b in range(_B):
        buf = _tc_ln(b, buf, gathered[b], pos, type_row, gamma, beta)
    return buf


# speedup vs baseline: 1.7646x; 1.1664x over previous
import functools

import jax
import jax.numpy as jnp
from jax import lax
from jax.experimental import pallas as pl
from jax.experimental.pallas import tpu as pltpu
from jax.experimental.pallas import tpu_sc as plsc

_B, _S, _H = 4, 2048, 768
_EPS = 1e-5
_NC, _NS = 2, 16
_NW = _NC * _NS
_CH = _S // _NW
_RB = 256
_NRB = _S // _RB


def _gather_kernel(ids_hbm, wemb_hbm, out_hbm, idx_v, buf_v, sem):
    wid = lax.axis_index("s") * _NC + lax.axis_index("c")
    base = wid * _CH
    pltpu.sync_copy(ids_hbm.at[pl.ds(base, _CH)], idx_v)
    pltpu.async_copy(wemb_hbm.at[idx_v], buf_v, sem).wait()
    pltpu.sync_copy(buf_v, out_hbm.at[pl.ds(base, _CH)])


def _sc_gather(ids_b, word_emb):
    mesh = plsc.VectorSubcoreMesh(core_axis_name="c", subcore_axis_name="s")
    run = pl.kernel(
        _gather_kernel,
        out_type=jax.ShapeDtypeStruct((_S, _H), jnp.float32),
        mesh=mesh,
        scratch_types=[
            pltpu.VMEM((_CH,), jnp.int32),
            pltpu.VMEM((_CH, _H), jnp.float32),
            pltpu.SemaphoreType.DMA,
        ],
    )
    return run(ids_b, word_emb)


def _ln_body(buf_ref, g_ref, p_ref, t_ref, gamma_ref, beta_ref, o_ref):
    x = g_ref[...] + p_ref[...] + t_ref[0][None, :]
    mean = jnp.mean(x, axis=-1, keepdims=True)
    xc = x - mean
    var = jnp.mean(xc * xc, axis=-1, keepdims=True)
    o_ref[0] = xc * lax.rsqrt(var + _EPS) * gamma_ref[...] + beta_ref[...]


def _tc_ln(b, buf, g_b, pos, type_row, gamma, beta):
    grid = (_NRB,)
    data_specs = [
        pl.BlockSpec((_RB, _H), lambda i: (i, 0)),
        pl.BlockSpec((_RB, _H), lambda i: (i, 0)),
        pl.BlockSpec((1, _H), lambda i: (0, 0)),
        pl.BlockSpec((_H,), lambda i: (0,)),
        pl.BlockSpec((_H,), lambda i: (0,)),
    ]
    out_spec = pl.BlockSpec((1, _RB, _H), lambda i, _b=b: (_b, i, 0))
    out_shape = jax.ShapeDtypeStruct((_B, _S, _H), jnp.float32)
    if buf is None:
        return pl.pallas_call(
            functools.partial(_ln_body, None),
            grid=grid,
            in_specs=data_specs,
            out_specs=out_spec,
            out_shape=out_shape,
        )(g_b, pos, type_row, gamma, beta)
    in_specs = [pl.BlockSpec((1, 8, 128), lambda i: (0, 0, 0))] + data_specs
    return pl.pallas_call(
        _ln_body,
        grid=grid,
        in_specs=in_specs,
        out_specs=out_spec,
        out_shape=out_shape,
        input_output_aliases={0: 0},
    )(buf, g_b, pos, type_row, gamma, beta)


def kernel(input_ids, word_emb, pos_emb, type_emb, gamma, beta):
    pos = pos_emb[:_S]
    type_row = type_emb.reshape(1, -1)[:, :_H]

    gathered = [_sc_gather(input_ids[b], word_emb) for b in range(_B)]

    buf = None
    for b in range(_B):
        buf = _tc_ln(b, buf, gathered[b], pos, type_row, gamma, beta)
    return buf


# speedup vs baseline: 1.8199x; 1.0313x over previous
import functools

import jax
import jax.numpy as jnp
from jax import lax
from jax.experimental import pallas as pl
from jax.experimental.pallas import tpu as pltpu
from jax.experimental.pallas import tpu_sc as plsc

_B, _S, _H = 4, 2048, 768
_EPS = 1e-5
_NC, _NS = 2, 16
_NW = _NC * _NS
_NCHUNK = 4
_CS = _S // _NCHUNK
_WP = _CS // _NW
_WR = _B * _WP
_RB = 256
_HB = _CS // _RB


def _gather_kernel(ids_hbm, wemb_hbm, out_hbm, idx_v, buf_v, semg, semg2, semw):
    wid = lax.axis_index("s") * _NC + lax.axis_index("c")
    base = wid * _WR
    pltpu.sync_copy(ids_hbm.at[pl.ds(base, _WR)], idx_v)
    half = _WR // 2
    pltpu.async_copy(
        wemb_hbm.at[idx_v.at[pl.ds(0, half)]], buf_v.at[pl.ds(0, half)], semg)
    pltpu.async_copy(
        wemb_hbm.at[idx_v.at[pl.ds(half, half)]],
        buf_v.at[pl.ds(half, half)], semg2)
    pltpu.make_async_copy(
        wemb_hbm.at[idx_v.at[pl.ds(0, half)]],
        buf_v.at[pl.ds(0, half)], semg).wait()
    for b in range(2):
        pltpu.async_copy(
            buf_v.at[pl.ds(b * _WP, _WP)],
            out_hbm.at[b, pl.ds(wid * _WP, _WP)], semw)
    pltpu.make_async_copy(
        wemb_hbm.at[idx_v.at[pl.ds(half, half)]],
        buf_v.at[pl.ds(half, half)], semg2).wait()
    for b in range(2, _B):
        pltpu.async_copy(
            buf_v.at[pl.ds(b * _WP, _WP)],
            out_hbm.at[b, pl.ds(wid * _WP, _WP)], semw)
    for b in range(_B):
        pltpu.make_async_copy(
            buf_v.at[pl.ds(b * _WP, _WP)],
            out_hbm.at[b, pl.ds(wid * _WP, _WP)], semw).wait()


def _sc_gather(ids_c, word_emb):
    mesh = plsc.VectorSubcoreMesh(core_axis_name="c", subcore_axis_name="s")
    run = pl.kernel(
        _gather_kernel,
        out_type=jax.ShapeDtypeStruct((_B, _CS, _H), jnp.float32),
        mesh=mesh,
        scratch_types=[
            pltpu.VMEM((_WR,), jnp.int32),
            pltpu.VMEM((_WR, _H), jnp.float32),
            pltpu.SemaphoreType.DMA,
            pltpu.SemaphoreType.DMA,
            pltpu.SemaphoreType.DMA,
        ],
    )
    return run(ids_c, word_emb)


def _ln_body(buf_ref, g_ref, p_ref, t_ref, gamma_ref, beta_ref, o_ref):
    x = g_ref[0] + p_ref[...] + t_ref[0][None, :]
    mean = jnp.mean(x, axis=-1, keepdims=True)
    xc = x - mean
    var = jnp.mean(xc * xc, axis=-1, keepdims=True)
    o_ref[0] = xc * lax.rsqrt(var + _EPS) * gamma_ref[...] + beta_ref[...]


def _tc_ln(c, buf, g_c, pos_emb, type_row, gamma, beta):
    grid = (_B * _HB,)
    data_specs = [
        pl.BlockSpec((1, _RB, _H), lambda i: (i // _HB, i % _HB, 0)),
        pl.BlockSpec((_RB, _H), lambda i, _c=c: (_c * _HB + i % _HB, 0)),
        pl.BlockSpec((1, _H), lambda i: (0, 0)),
        pl.BlockSpec((_H,), lambda i: (0,)),
        pl.BlockSpec((_H,), lambda i: (0,)),
    ]
    out_spec = pl.BlockSpec(
        (1, _RB, _H), lambda i, _c=c: (i // _HB, _c * _HB + i % _HB, 0))
    out_shape = jax.ShapeDtypeStruct((_B, _S, _H), jnp.float32)
    if buf is None:
        return pl.pallas_call(
            functools.partial(_ln_body, None),
            grid=grid,
            in_specs=data_specs,
            out_specs=out_spec,
            out_shape=out_shape,
        )(g_c, pos_emb, type_row, gamma, beta)
    in_specs = [pl.BlockSpec((1, 8, 128), lambda i: (0, 0, 0))] + data_specs
    return pl.pallas_call(
        _ln_body,
        grid=grid,
        in_specs=in_specs,
        out_specs=out_spec,
        out_shape=out_shape,
        input_output_aliases={0: 0},
    )(buf, g_c, pos_emb, type_row, gamma, beta)


def kernel(input_ids, word_emb, pos_emb, type_emb, gamma, beta):
    type_row = type_emb.reshape(1, -1)[:, :_H]
    ids_re = (input_ids.reshape(_B, _NCHUNK, _NW, _WP)
              .transpose(1, 2, 0, 3).reshape(_NCHUNK, _NW * _WR))

    gathered = [_sc_gather(ids_re[c], word_emb) for c in range(_NCHUNK)]

    buf = None
    for c in range(_NCHUNK):
        buf = _tc_ln(c, buf, gathered[c], pos_emb, type_row, gamma, beta)
    return buf
